# baseline (speedup 1.0000x reference)
import jax
import jax.numpy as jnp
from jax import lax
from jax.experimental import pallas as pl
from jax.experimental.pallas import tpu as pltpu
from jax.experimental.pallas import tpu_sc as plsc

_NC = 2
_NS = 16
_NW = _NC * _NS
_B = 4096
_PW = _B // _NW
_D = 128
_CK = 4
_CP = _PW // _CK
_MARGIN = 1.0


def _rsqrt(x):
    i = lax.bitcast_convert_type(x, jnp.int32)
    i = jnp.int32(0x5F3759DF) - lax.shift_right_logical(i, 1)
    y = lax.bitcast_convert_type(i, jnp.float32)
    for _ in range(3):
        y = y * (jnp.float32(1.5) - jnp.float32(0.5) * x * y * y)
    return y


def _body(ent, rel, idx_hbm, out, ixf, rpe, rne, rr, lv,
          sem0, sem1, sem2, sem3):
    wid = lax.axis_index("s") * _NC + lax.axis_index("c")

    pltpu.sync_copy(idx_hbm.at[wid], ixf)

    sems = (sem0, sem1, sem2, sem3)
    copies = []
    for c in range(_CK):
        sl = pl.ds(c * 2 * _CP, 2 * _CP)
        copies.append((
            pltpu.async_copy(ent.at[ixf.at[c, 0]], rpe.at[sl], sems[c]),
            pltpu.async_copy(ent.at[ixf.at[c, 1]], rne.at[sl], sems[c]),
            pltpu.async_copy(rel.at[ixf.at[c, 2]], rr.at[sl], sems[c]),
        ))

    lane = lax.iota(jnp.int32, 16)
    zero = jnp.zeros((16,), jnp.float32)
    two = jnp.float32(2.0)
    eps_n = jnp.float32(1e-24)
    eps_d = jnp.float32(1e-12)

    loss = zero
    for c in range(_CK):
        for cp in copies[c]:
            cp.wait()
        for gg in range(_CP // 16):
            base = jnp.int32(c * 2 * _CP + gg * 16)
            r_ph = lane + base
            r_pt = r_ph + jnp.int32(_CP)

            def dim_body(d, acc):
                (psh, psr, pst, pshr, psht, psrt,
                 nsh, nsr, nst, nshr, nsht, nsrt) = acc
                dv = (jnp.full((16,), d, jnp.int32) + lane) \
                    & jnp.int32(_D - 1)
                hh = plsc.load_gather(rpe, [r_ph, dv])
                rv = plsc.load_gather(rr, [r_ph, dv])
                tt = plsc.load_gather(rpe, [r_pt, dv])
                psh = psh + hh * hh
                psr = psr + rv * rv
                pst = pst + tt * tt
                pshr = pshr + hh * rv
                psht = psht + hh * tt
                psrt = psrt + rv * tt
                hh = plsc.load_gather(rne, [r_ph, dv])
                rv = plsc.load_gather(rr, [r_pt, dv])
                tt = plsc.load_gather(rne, [r_pt, dv])
                nsh = nsh + hh * hh
                nsr = nsr + rv * rv
                nst = nst + tt * tt
                nshr = nshr + hh * rv
                nsht = nsht + hh * tt
                nsrt = nsrt + rv * tt
                return (psh, psr, pst, pshr, psht, psrt,
                        nsh, nsr, nst, nshr, nsht, nsrt)

            (psh, psr, pst, pshr, psht, psrt,
             nsh, nsr, nst, nshr, nsht, nsrt) = lax.fori_loop(
                0, _D, dim_body, (zero,) * 12, unroll=8)

            ih = _rsqrt(jnp.maximum(psh, eps_n))
            it = _rsqrt(jnp.maximum(pst, eps_n))
            sp = (psh * ih * ih + psr + pst * it * it
                  + two * (ih * pshr - ih * it * psht - it * psrt)) + eps_d
            ih = _rsqrt(jnp.maximum(nsh, eps_n))
            it = _rsqrt(jnp.maximum(nst, eps_n))
            sn = (nsh * ih * ih + nsr + nst * it * it
                  + two * (ih * nshr - ih * it * nsht - it * nsrt)) + eps_d
            dp = sp * _rsqrt(sp)
            dn = sn * _rsqrt(sn)
            loss = loss + jnp.maximum(dp - dn + jnp.float32(_MARGIN),
                                      jnp.float32(0.0))

    lv[...] = loss
    pltpu.sync_copy(lv, out.at[wid])


@jax.jit
def _transe_loss(entity_emb, relation_emb, idx_all):
    mesh = plsc.VectorSubcoreMesh(core_axis_name="c", subcore_axis_name="s")
    f = pl.kernel(
        _body,
        out_type=jax.ShapeDtypeStruct((_NW, 16), jnp.float32),
        mesh=mesh,
        compiler_params=pltpu.CompilerParams(needs_layout_passes=False),
        scratch_types=[
            pltpu.VMEM((_CK, 3, 2 * _CP), jnp.int32),
            pltpu.VMEM((2 * _PW, _D), jnp.float32),
            pltpu.VMEM((2 * _PW, _D), jnp.float32),
            pltpu.VMEM((2 * _PW, _D), jnp.float32),
            pltpu.VMEM((16,), jnp.float32),
            pltpu.SemaphoreType.DMA,
            pltpu.SemaphoreType.DMA,
            pltpu.SemaphoreType.DMA,
            pltpu.SemaphoreType.DMA,
        ],
    )
    partials = f(entity_emb, relation_emb, idx_all)
    return jnp.sum(partials)


def kernel(entity_emb, relation_emb, pos_heads, pos_rels, pos_tails,
           neg_heads, neg_rels, neg_tails):
    ph, pr, pt, nh, nr, nt = (
        x.astype(jnp.int32).reshape(_NW, _CK, _CP)
        for x in (pos_heads, pos_rels, pos_tails,
                  neg_heads, neg_rels, neg_tails))
    s0 = jnp.concatenate([ph, pt], axis=-1)
    s1 = jnp.concatenate([nh, nt], axis=-1)
    s2 = jnp.concatenate([pr, nr], axis=-1)
    idx_all = jnp.stack([s0, s1, s2], axis=2)
    return _transe_loss(entity_emb, relation_emb, idx_all)


# speedup vs baseline: 1.0724x; 1.0724x over previous
import jax
import jax.numpy as jnp
from jax import lax
from jax.experimental import pallas as pl
from jax.experimental.pallas import tpu as pltpu
from jax.experimental.pallas import tpu_sc as plsc

_NC = 2
_NS = 16
_NW = _NC * _NS
_B = 4096
_PW = _B // _NW
_D = 128
_CK = 4
_CP = _PW // _CK
_MARGIN = 1.0


def _rsqrt(x):
    i = lax.bitcast_convert_type(x, jnp.int32)
    i = jnp.int32(0x5F3759DF) - lax.shift_right_logical(i, 1)
    y = lax.bitcast_convert_type(i, jnp.float32)
    for _ in range(3):
        y = y * (jnp.float32(1.5) - jnp.float32(0.5) * x * y * y)
    return y


def _body(ent, rel, iph_h, ipr_h, ipt_h, inh_h, inr_h, int_h, out,
          iph, ipr, ipt, inh, inr, itn, rpe, rne, rr, lv,
          sem_i, sem0, sem1, sem2, sem3):
    wid = lax.axis_index("s") * _NC + lax.axis_index("c")
    base = wid * _PW

    stage = [
        pltpu.async_copy(iph_h.at[pl.ds(base, _PW)], iph, sem_i),
        pltpu.async_copy(ipr_h.at[pl.ds(base, _PW)], ipr, sem_i),
        pltpu.async_copy(ipt_h.at[pl.ds(base, _PW)], ipt, sem_i),
        pltpu.async_copy(inh_h.at[pl.ds(base, _PW)], inh, sem_i),
        pltpu.async_copy(inr_h.at[pl.ds(base, _PW)], inr, sem_i),
        pltpu.async_copy(int_h.at[pl.ds(base, _PW)], itn, sem_i),
    ]
    for c in stage:
        c.wait()

    sems = (sem0, sem1, sem2, sem3)
    copies = []
    for c in range(_CK):
        isl = pl.ds(c * _CP, _CP)
        psl = pl.ds(c * _CP, _CP)
        nsl = pl.ds(_PW + c * _CP, _CP)
        copies.append((
            pltpu.async_copy(ent.at[iph.at[isl]], rpe.at[psl], sems[c]),
            pltpu.async_copy(ent.at[ipt.at[isl]], rne.at[psl], sems[c]),
            pltpu.async_copy(rel.at[ipr.at[isl]], rr.at[psl], sems[c]),
            pltpu.async_copy(ent.at[inh.at[isl]], rpe.at[nsl], sems[c]),
            pltpu.async_copy(ent.at[itn.at[isl]], rne.at[nsl], sems[c]),
            pltpu.async_copy(rel.at[inr.at[isl]], rr.at[nsl], sems[c]),
        ))

    lane = lax.iota(jnp.int32, 16)
    zero = jnp.zeros((16,), jnp.float32)
    two = jnp.float32(2.0)
    eps_n = jnp.float32(1e-24)
    eps_d = jnp.float32(1e-12)

    loss = zero
    for c in range(_CK):
        for cp in copies[c]:
            cp.wait()
        for gg in range(_CP // 16):
            pbase = lane + jnp.int32(c * _CP + gg * 16)
            nbase = pbase + jnp.int32(_PW)

            def dim_body(d, acc):
                (psh, psr, pst, pshr, psht, psrt,
                 nsh, nsr, nst, nshr, nsht, nsrt) = acc
                dv = (jnp.full((16,), d, jnp.int32) + lane) \
                    & jnp.int32(_D - 1)
                hh = plsc.load_gather(rpe, [pbase, dv])
                rv = plsc.load_gather(rr, [pbase, dv])
                tt = plsc.load_gather(rne, [pbase, dv])
                psh = psh + hh * hh
                psr = psr + rv * rv
                pst = pst + tt * tt
                pshr = pshr + hh * rv
                psht = psht + hh * tt
                psrt = psrt + rv * tt
                hh = plsc.load_gather(rpe, [nbase, dv])
                rv = plsc.load_gather(rr, [nbase, dv])
                tt = plsc.load_gather(rne, [nbase, dv])
                nsh = nsh + hh * hh
                nsr = nsr + rv * rv
                nst = nst + tt * tt
                nshr = nshr + hh * rv
                nsht = nsht + hh * tt
                nsrt = nsrt + rv * tt
                return (psh, psr, pst, pshr, psht, psrt,
                        nsh, nsr, nst, nshr, nsht, nsrt)

            (psh, psr, pst, pshr, psht, psrt,
             nsh, nsr, nst, nshr, nsht, nsrt) = lax.fori_loop(
                0, _D, dim_body, (zero,) * 12, unroll=8)

            ih = _rsqrt(jnp.maximum(psh, eps_n))
            it = _rsqrt(jnp.maximum(pst, eps_n))
            sp = (psh * ih * ih + psr + pst * it * it
                  + two * (ih * pshr - ih * it * psht - it * psrt)) + eps_d
            ih = _rsqrt(jnp.maximum(nsh, eps_n))
            it = _rsqrt(jnp.maximum(nst, eps_n))
            sn = (nsh * ih * ih + nsr + nst * it * it
                  + two * (ih * nshr - ih * it * nsht - it * nsrt)) + eps_d
            dp = sp * _rsqrt(sp)
            dn = sn * _rsqrt(sn)
            loss = loss + jnp.maximum(dp - dn + jnp.float32(_MARGIN),
                                      jnp.float32(0.0))

    lv[...] = loss
    pltpu.sync_copy(lv, out.at[wid])


@jax.jit
def _transe_loss(entity_emb, relation_emb, iph, ipr, ipt, inh, inr, itn):
    mesh = plsc.VectorSubcoreMesh(core_axis_name="c", subcore_axis_name="s")
    f = pl.kernel(
        _body,
        out_type=jax.ShapeDtypeStruct((_NW, 16), jnp.float32),
        mesh=mesh,
        compiler_params=pltpu.CompilerParams(needs_layout_passes=False),
        scratch_types=[
            pltpu.VMEM((_PW,), jnp.int32),
            pltpu.VMEM((_PW,), jnp.int32),
            pltpu.VMEM((_PW,), jnp.int32),
            pltpu.VMEM((_PW,), jnp.int32),
            pltpu.VMEM((_PW,), jnp.int32),
            pltpu.VMEM((_PW,), jnp.int32),
            pltpu.VMEM((2 * _PW, _D), jnp.float32),
            pltpu.VMEM((2 * _PW, _D), jnp.float32),
            pltpu.VMEM((2 * _PW, _D), jnp.float32),
            pltpu.VMEM((16,), jnp.float32),
            pltpu.SemaphoreType.DMA,
            pltpu.SemaphoreType.DMA,
            pltpu.SemaphoreType.DMA,
            pltpu.SemaphoreType.DMA,
            pltpu.SemaphoreType.DMA,
        ],
    )
    partials = f(entity_emb, relation_emb, iph, ipr, ipt, inh, inr, itn)
    return jnp.sum(partials)


def kernel(entity_emb, relation_emb, pos_heads, pos_rels, pos_tails,
           neg_heads, neg_rels, neg_tails):
    idx = [x.astype(jnp.int32) for x in (pos_heads, pos_rels, pos_tails,
                                         neg_heads, neg_rels, neg_tails)]
    return _transe_loss(entity_emb, relation_emb, *idx)


# speedup vs baseline: 1.1318x; 1.0553x over previous
import jax
import jax.numpy as jnp
from jax import lax
from jax.experimental import pallas as pl
from jax.experimental.pallas import tpu as pltpu
from jax.experimental.pallas import tpu_sc as plsc

_NC = 2
_NS = 16
_NW = _NC * _NS
_B = 4096
_PW = _B // _NW
_D = 128
_H = _PW // 2
_MARGIN = 1.0


def _rsqrt(x):
    i = lax.bitcast_convert_type(x, jnp.int32)
    i = jnp.int32(0x5F3759DF) - lax.shift_right_logical(i, 1)
    y = lax.bitcast_convert_type(i, jnp.float32)
    for _ in range(3):
        y = y * (jnp.float32(1.5) - jnp.float32(0.5) * x * y * y)
    return y


def _body(ent, rel, iph_h, ipr_h, ipt_h, inh_h, inr_h, int_h, out,
          iph, ipr, ipt, inh, inr, itn, rpe, rne, rr, lv, sem_i,
          sem0, sem1):
    wid = lax.axis_index("s") * _NC + lax.axis_index("c")
    base = wid * _PW

    stage = [
        pltpu.async_copy(iph_h.at[pl.ds(base, _PW)], iph, sem_i),
        pltpu.async_copy(ipr_h.at[pl.ds(base, _PW)], ipr, sem_i),
        pltpu.async_copy(ipt_h.at[pl.ds(base, _PW)], ipt, sem_i),
        pltpu.async_copy(inh_h.at[pl.ds(base, _PW)], inh, sem_i),
        pltpu.async_copy(inr_h.at[pl.ds(base, _PW)], inr, sem_i),
        pltpu.async_copy(int_h.at[pl.ds(base, _PW)], itn, sem_i),
    ]
    for c in stage:
        c.wait()

    sems = (sem0, sem1)
    copies = []
    for h in range(2):
        isl = pl.ds(h * _H, _H)
        psl = pl.ds(h * _H, _H)
        nsl = pl.ds(_PW + h * _H, _H)
        copies.append((
            pltpu.async_copy(ent.at[iph.at[isl]], rpe.at[psl], sems[h]),
            pltpu.async_copy(ent.at[ipt.at[isl]], rne.at[psl], sems[h]),
            pltpu.async_copy(rel.at[ipr.at[isl]], rr.at[psl], sems[h]),
            pltpu.async_copy(ent.at[inh.at[isl]], rpe.at[nsl], sems[h]),
            pltpu.async_copy(ent.at[itn.at[isl]], rne.at[nsl], sems[h]),
            pltpu.async_copy(rel.at[inr.at[isl]], rr.at[nsl], sems[h]),
        ))

    lane = lax.iota(jnp.int32, 16)
    zero = jnp.zeros((16,), jnp.float32)
    two = jnp.float32(2.0)
    eps_n = jnp.float32(1e-24)
    eps_d = jnp.float32(1e-12)

    def dim_body(d, acc):
        (pbase, nbase,
         psh, psr, pst, pshr, psht, psrt,
         nsh, nsr, nst, nshr, nsht, nsrt) = acc
        dv = (jnp.full((16,), d, jnp.int32) + lane) & jnp.int32(_D - 1)
        hh = plsc.load_gather(rpe, [pbase, dv])
        rv = plsc.load_gather(rr, [pbase, dv])
        tt = plsc.load_gather(rne, [pbase, dv])
        psh = psh + hh * hh
        psr = psr + rv * rv
        pst = pst + tt * tt
        pshr = pshr + hh * rv
        psht = psht + hh * tt
        psrt = psrt + rv * tt
        hh = plsc.load_gather(rpe, [nbase, dv])
        rv = plsc.load_gather(rr, [nbase, dv])
        tt = plsc.load_gather(rne, [nbase, dv])
        nsh = nsh + hh * hh
        nsr = nsr + rv * rv
        nst = nst + tt * tt
        nshr = nshr + hh * rv
        nsht = nsht + hh * tt
        nsrt = nsrt + rv * tt
        return (pbase, nbase,
                psh, psr, pst, pshr, psht, psrt,
                nsh, nsr, nst, nshr, nsht, nsrt)

    def make_group_body(h):
        def group_body(gg, loss):
            pbase = lane + jnp.int32(h * _H) + gg * 16
            nbase = pbase + jnp.int32(_PW)
            acc = lax.fori_loop(0, _D, dim_body,
                                (pbase, nbase) + (zero,) * 12, unroll=8)
            (psh, psr, pst, pshr, psht, psrt,
             nsh, nsr, nst, nshr, nsht, nsrt) = acc[2:]

            ih = _rsqrt(jnp.maximum(psh, eps_n))
            it = _rsqrt(jnp.maximum(pst, eps_n))
            sp = (psh * ih * ih + psr + pst * it * it
                  + two * (ih * pshr - ih * it * psht - it * psrt)) + eps_d
            ih = _rsqrt(jnp.maximum(nsh, eps_n))
            it = _rsqrt(jnp.maximum(nst, eps_n))
            sn = (nsh * ih * ih + nsr + nst * it * it
                  + two * (ih * nshr - ih * it * nsht - it * nsrt)) + eps_d
            dp = sp * _rsqrt(sp)
            dn = sn * _rsqrt(sn)
            return loss + jnp.maximum(dp - dn + jnp.float32(_MARGIN),
                                      jnp.float32(0.0))
        return group_body

    loss = zero
    for h in range(2):
        for cp in copies[h]:
            cp.wait()
        loss = lax.fori_loop(0, _H // 16, make_group_body(h), loss)

    lv[...] = loss
    pltpu.sync_copy(lv, out.at[wid])


@jax.jit
def _transe_loss(entity_emb, relation_emb, iph, ipr, ipt, inh, inr, itn):
    mesh = plsc.VectorSubcoreMesh(core_axis_name="c", subcore_axis_name="s")
    f = pl.kernel(
        _body,
        out_type=jax.ShapeDtypeStruct((_NW, 16), jnp.float32),
        mesh=mesh,
        compiler_params=pltpu.CompilerParams(needs_layout_passes=False),
        scratch_types=[
            pltpu.VMEM((_PW,), jnp.int32),
            pltpu.VMEM((_PW,), jnp.int32),
            pltpu.VMEM((_PW,), jnp.int32),
            pltpu.VMEM((_PW,), jnp.int32),
            pltpu.VMEM((_PW,), jnp.int32),
            pltpu.VMEM((_PW,), jnp.int32),
            pltpu.VMEM((2 * _PW, _D), jnp.float32),
            pltpu.VMEM((2 * _PW, _D), jnp.float32),
            pltpu.VMEM((2 * _PW, _D), jnp.float32),
            pltpu.VMEM((16,), jnp.float32),
            pltpu.SemaphoreType.DMA,
            pltpu.SemaphoreType.DMA,
            pltpu.SemaphoreType.DMA,
        ],
    )
    partials = f(entity_emb, relation_emb, iph, ipr, ipt, inh, inr, itn)
    return jnp.sum(partials)


def kernel(entity_emb, relation_emb, pos_heads, pos_rels, pos_tails,
           neg_heads, neg_rels, neg_tails):
    idx = [x.astype(jnp.int32) for x in (pos_heads, pos_rels, pos_tails,
                                         neg_heads, neg_rels, neg_tails)]
    return _transe_loss(entity_emb, relation_emb, *idx)


# speedup vs baseline: 1.1468x; 1.0133x over previous
import jax
import jax.numpy as jnp
from jax import lax
from jax.experimental import pallas as pl
from jax.experimental.pallas import tpu as pltpu
from jax.experimental.pallas import tpu_sc as plsc

_NC = 2
_NS = 16
_NW = _NC * _NS
_B = 4096
_PW = _B // _NW
_D = 128
_CK = 4
_CP = _PW // _CK
_MARGIN = 1.0


def _rsqrt(x):
    i = lax.bitcast_convert_type(x, jnp.int32)
    i = jnp.int32(0x5F3759DF) - lax.shift_right_logical(i, 1)
    y = lax.bitcast_convert_type(i, jnp.float32)
    for _ in range(3):
        y = y * (jnp.float32(1.5) - jnp.float32(0.5) * x * y * y)
    return y


def _body(ent, rel, iph_h, ipr_h, ipt_h, inh_h, inr_h, int_h, out,
          iph, ipr, ipt, inh, inr, itn, rpe, rne, rr, lv, sem_i,
          sem0, sem1, sem2, sem3):
    wid = lax.axis_index("s") * _NC + lax.axis_index("c")
    base = wid * _PW

    stage = [
        pltpu.async_copy(iph_h.at[pl.ds(base, _PW)], iph, sem_i),
        pltpu.async_copy(ipr_h.at[pl.ds(base, _PW)], ipr, sem_i),
        pltpu.async_copy(ipt_h.at[pl.ds(base, _PW)], ipt, sem_i),
        pltpu.async_copy(inh_h.at[pl.ds(base, _PW)], inh, sem_i),
        pltpu.async_copy(inr_h.at[pl.ds(base, _PW)], inr, sem_i),
        pltpu.async_copy(int_h.at[pl.ds(base, _PW)], itn, sem_i),
    ]
    for c in stage:
        c.wait()

    sems = (sem0, sem1, sem2, sem3)
    copies = []
    for ch in range(_CK):
        isl = pl.ds(ch * _CP, _CP)
        psl = pl.ds(ch * _CP, _CP)
        nsl = pl.ds(_PW + ch * _CP, _CP)
        copies.append((
            pltpu.async_copy(ent.at[iph.at[isl]], rpe.at[psl], sems[ch]),
            pltpu.async_copy(ent.at[ipt.at[isl]], rne.at[psl], sems[ch]),
            pltpu.async_copy(rel.at[ipr.at[isl]], rr.at[psl], sems[ch]),
            pltpu.async_copy(ent.at[inh.at[isl]], rpe.at[nsl], sems[ch]),
            pltpu.async_copy(ent.at[itn.at[isl]], rne.at[nsl], sems[ch]),
            pltpu.async_copy(rel.at[inr.at[isl]], rr.at[nsl], sems[ch]),
        ))

    lane = lax.iota(jnp.int32, 16)
    zero = jnp.zeros((16,), jnp.float32)
    two = jnp.float32(2.0)
    eps_n = jnp.float32(1e-24)
    eps_d = jnp.float32(1e-12)

    def dim_body(d, acc):
        (pbase, nbase,
         psh, psr, pst, pshr, psht, psrt,
         nsh, nsr, nst, nshr, nsht, nsrt) = acc
        dv = (jnp.full((16,), d, jnp.int32) + lane) & jnp.int32(_D - 1)
        hh = plsc.load_gather(rpe, [pbase, dv])
        rv = plsc.load_gather(rr, [pbase, dv])
        tt = plsc.load_gather(rne, [pbase, dv])
        psh = psh + hh * hh
        psr = psr + rv * rv
        pst = pst + tt * tt
        pshr = pshr + hh * rv
        psht = psht + hh * tt
        psrt = psrt + rv * tt
        hh = plsc.load_gather(rpe, [nbase, dv])
        rv = plsc.load_gather(rr, [nbase, dv])
        tt = plsc.load_gather(rne, [nbase, dv])
        nsh = nsh + hh * hh
        nsr = nsr + rv * rv
        nst = nst + tt * tt
        nshr = nshr + hh * rv
        nsht = nsht + hh * tt
        nsrt = nsrt + rv * tt
        return (pbase, nbase,
                psh, psr, pst, pshr, psht, psrt,
                nsh, nsr, nst, nshr, nsht, nsrt)

    def group_body(gg, loss):
        for ch in range(_CK):
            @pl.when(gg == jnp.int32(ch * 2))
            def _wait():
                for cp in copies[ch]:
                    cp.wait()

        if True:
            pbase = lane + gg * 16
            nbase = pbase + jnp.int32(_PW)
            acc = lax.fori_loop(0, _D, dim_body,
                                (pbase, nbase) + (zero,) * 12, unroll=8)
            (psh, psr, pst, pshr, psht, psrt,
             nsh, nsr, nst, nshr, nsht, nsrt) = acc[2:]

            ih = _rsqrt(jnp.maximum(psh, eps_n))
            it = _rsqrt(jnp.maximum(pst, eps_n))
            sp = (psh * ih * ih + psr + pst * it * it
                  + two * (ih * pshr - ih * it * psht - it * psrt)) + eps_d
            ih = _rsqrt(jnp.maximum(nsh, eps_n))
            it = _rsqrt(jnp.maximum(nst, eps_n))
            sn = (nsh * ih * ih + nsr + nst * it * it
                  + two * (ih * nshr - ih * it * nsht - it * nsrt)) + eps_d
            dp = sp * _rsqrt(sp)
            dn = sn * _rsqrt(sn)
            return loss + jnp.maximum(dp - dn + jnp.float32(_MARGIN),
                                      jnp.float32(0.0))

    loss = lax.fori_loop(0, _PW // 16, group_body, zero)

    lv[...] = loss
    pltpu.sync_copy(lv, out.at[wid])


@jax.jit
def _transe_loss(entity_emb, relation_emb, iph, ipr, ipt, inh, inr, itn):
    mesh = plsc.VectorSubcoreMesh(core_axis_name="c", subcore_axis_name="s")
    f = pl.kernel(
        _body,
        out_type=jax.ShapeDtypeStruct((_NW, 16), jnp.float32),
        mesh=mesh,
        compiler_params=pltpu.CompilerParams(needs_layout_passes=False),
        scratch_types=[
            pltpu.VMEM((_PW,), jnp.int32),
            pltpu.VMEM((_PW,), jnp.int32),
            pltpu.VMEM((_PW,), jnp.int32),
            pltpu.VMEM((_PW,), jnp.int32),
            pltpu.VMEM((_PW,), jnp.int32),
            pltpu.VMEM((_PW,), jnp.int32),
            pltpu.VMEM((2 * _PW, _D), jnp.float32),
            pltpu.VMEM((2 * _PW, _D), jnp.float32),
            pltpu.VMEM((2 * _PW, _D), jnp.float32),
            pltpu.VMEM((16,), jnp.float32),
            pltpu.SemaphoreType.DMA,
            pltpu.SemaphoreType.DMA,
            pltpu.SemaphoreType.DMA,
            pltpu.SemaphoreType.DMA,
            pltpu.SemaphoreType.DMA,
        ],
    )
    partials = f(entity_emb, relation_emb, iph, ipr, ipt, inh, inr, itn)
    return jnp.sum(partials)


def kernel(entity_emb, relation_emb, pos_heads, pos_rels, pos_tails,
           neg_heads, neg_rels, neg_tails):
    idx = [x.astype(jnp.int32) for x in (pos_heads, pos_rels, pos_tails,
                                         neg_heads, neg_rels, neg_tails)]
    return _transe_loss(entity_emb, relation_emb, *idx)


# speedup vs baseline: 1.1660x; 1.0167x over previous
import jax
import jax.numpy as jnp
from jax import lax
from jax.experimental import pallas as pl
from jax.experimental.pallas import tpu as pltpu
from jax.experimental.pallas import tpu_sc as plsc

_NC = 2
_NS = 16
_NW = _NC * _NS
_B = 4096
_PW = _B // _NW
_D = 128
_CK = 4
_CHUNKS = ((0, 16), (16, 16), (32, 32), (64, 64))
_WAIT_AT = (0, 1, 2, 4)
_MARGIN = 1.0


def _rsqrt(x):
    i = lax.bitcast_convert_type(x, jnp.int32)
    i = jnp.int32(0x5F3759DF) - lax.shift_right_logical(i, 1)
    y = lax.bitcast_convert_type(i, jnp.float32)
    for _ in range(3):
        y = y * (jnp.float32(1.5) - jnp.float32(0.5) * x * y * y)
    return y


def _body(ent, rel, iph_h, ipr_h, ipt_h, inh_h, inr_h, int_h, out,
          iph, ipr, ipt, inh, inr, itn, rpe, rne, rr, lv, sem_i,
          sem0, sem1, sem2, sem3):
    wid = lax.axis_index("s") * _NC + lax.axis_index("c")
    base = wid * _PW

    stage = [
        pltpu.async_copy(iph_h.at[pl.ds(base, _PW)], iph, sem_i),
        pltpu.async_copy(ipr_h.at[pl.ds(base, _PW)], ipr, sem_i),
        pltpu.async_copy(ipt_h.at[pl.ds(base, _PW)], ipt, sem_i),
        pltpu.async_copy(inh_h.at[pl.ds(base, _PW)], inh, sem_i),
        pltpu.async_copy(inr_h.at[pl.ds(base, _PW)], inr, sem_i),
        pltpu.async_copy(int_h.at[pl.ds(base, _PW)], itn, sem_i),
    ]
    for c in stage:
        c.wait()

    sems = (sem0, sem1, sem2, sem3)
    copies = []
    for ch in range(_CK):
        start, size = _CHUNKS[ch]
        isl = pl.ds(start, size)
        psl = pl.ds(start, size)
        nsl = pl.ds(_PW + start, size)
        copies.append((
            pltpu.async_copy(ent.at[iph.at[isl]], rpe.at[psl], sems[ch]),
            pltpu.async_copy(ent.at[ipt.at[isl]], rne.at[psl], sems[ch]),
            pltpu.async_copy(rel.at[ipr.at[isl]], rr.at[psl], sems[ch]),
            pltpu.async_copy(ent.at[inh.at[isl]], rpe.at[nsl], sems[ch]),
            pltpu.async_copy(ent.at[itn.at[isl]], rne.at[nsl], sems[ch]),
            pltpu.async_copy(rel.at[inr.at[isl]], rr.at[nsl], sems[ch]),
        ))

    lane = lax.iota(jnp.int32, 16)
    zero = jnp.zeros((16,), jnp.float32)
    two = jnp.float32(2.0)
    eps_n = jnp.float32(1e-24)
    eps_d = jnp.float32(1e-12)

    def dim_body(d, acc):
        (pbase, nbase,
         psh, psr, pst, pshr, psht, psrt,
         nsh, nsr, nst, nshr, nsht, nsrt) = acc
        dv = (jnp.full((16,), d, jnp.int32) + lane) & jnp.int32(_D - 1)
        hh = plsc.load_gather(rpe, [pbase, dv])
        rv = plsc.load_gather(rr, [pbase, dv])
        tt = plsc.load_gather(rne, [pbase, dv])
        psh = psh + hh * hh
        psr = psr + rv * rv
        pst = pst + tt * tt
        pshr = pshr + hh * rv
        psht = psht + hh * tt
        psrt = psrt + rv * tt
        hh = plsc.load_gather(rpe, [nbase, dv])
        rv = plsc.load_gather(rr, [nbase, dv])
        tt = plsc.load_gather(rne, [nbase, dv])
        nsh = nsh + hh * hh
        nsr = nsr + rv * rv
        nst = nst + tt * tt
        nshr = nshr + hh * rv
        nsht = nsht + hh * tt
        nsrt = nsrt + rv * tt
        return (pbase, nbase,
                psh, psr, pst, pshr, psht, psrt,
                nsh, nsr, nst, nshr, nsht, nsrt)

    def group_body(gg, loss):
        for ch in range(_CK):
            @pl.when(gg == jnp.int32(_WAIT_AT[ch]))
            def _wait():
                for cp in copies[ch]:
                    cp.wait()

        if True:
            pbase = lane + gg * 16
            nbase = pbase + jnp.int32(_PW)
            acc = lax.fori_loop(0, _D, dim_body,
                                (pbase, nbase) + (zero,) * 12, unroll=8)
            (psh, psr, pst, pshr, psht, psrt,
             nsh, nsr, nst, nshr, nsht, nsrt) = acc[2:]

            ih = _rsqrt(jnp.maximum(psh, eps_n))
            it = _rsqrt(jnp.maximum(pst, eps_n))
            sp = (psh * ih * ih + psr + pst * it * it
                  + two * (ih * pshr - ih * it * psht - it * psrt)) + eps_d
            ih = _rsqrt(jnp.maximum(nsh, eps_n))
            it = _rsqrt(jnp.maximum(nst, eps_n))
            sn = (nsh * ih * ih + nsr + nst * it * it
                  + two * (ih * nshr - ih * it * nsht - it * nsrt)) + eps_d
            dp = sp * _rsqrt(sp)
            dn = sn * _rsqrt(sn)
            return loss + jnp.maximum(dp - dn + jnp.float32(_MARGIN),
                                      jnp.float32(0.0))

    loss = lax.fori_loop(0, _PW // 16, group_body, zero)

    lv[...] = loss
    pltpu.sync_copy(lv, out.at[wid])


@jax.jit
def _transe_loss(entity_emb, relation_emb, iph, ipr, ipt, inh, inr, itn):
    mesh = plsc.VectorSubcoreMesh(core_axis_name="c", subcore_axis_name="s")
    f = pl.kernel(
        _body,
        out_type=jax.ShapeDtypeStruct((_NW, 16), jnp.float32),
        mesh=mesh,
        compiler_params=pltpu.CompilerParams(needs_layout_passes=False),
        scratch_types=[
            pltpu.VMEM((_PW,), jnp.int32),
            pltpu.VMEM((_PW,), jnp.int32),
            pltpu.VMEM((_PW,), jnp.int32),
            pltpu.VMEM((_PW,), jnp.int32),
            pltpu.VMEM((_PW,), jnp.int32),
            pltpu.VMEM((_PW,), jnp.int32),
            pltpu.VMEM((2 * _PW, _D), jnp.float32),
            pltpu.VMEM((2 * _PW, _D), jnp.float32),
            pltpu.VMEM((2 * _PW, _D), jnp.float32),
            pltpu.VMEM((16,), jnp.float32),
            pltpu.SemaphoreType.DMA,
            pltpu.SemaphoreType.DMA,
            pltpu.SemaphoreType.DMA,
            pltpu.SemaphoreType.DMA,
            pltpu.SemaphoreType.DMA,
        ],
    )
    partials = f(entity_emb, relation_emb, iph, ipr, ipt, inh, inr, itn)
    return jnp.sum(partials)


def kernel(entity_emb, relation_emb, pos_heads, pos_rels, pos_tails,
           neg_heads, neg_rels, neg_tails):
    idx = [x.astype(jnp.int32) for x in (pos_heads, pos_rels, pos_tails,
                                         neg_heads, neg_rels, neg_tails)]
    return _transe_loss(entity_emb, relation_emb, *idx)


# speedup vs baseline: 1.1927x; 1.0230x over previous
import jax
import jax.numpy as jnp
from jax import lax
from jax.experimental import pallas as pl
from jax.experimental.pallas import tpu as pltpu
from jax.experimental.pallas import tpu_sc as plsc

_NC = 2
_NS = 16
_NW = _NC * _NS
_B = 4096
_PW = _B // _NW
_D = 128
_CK = 4
_CHUNKS = ((0, 16), (16, 16), (32, 32), (64, 64))
_WAIT_AT = (0, 1, 2, 4)
_MARGIN = 1.0


def _rsqrt(x):
    i = lax.bitcast_convert_type(x, jnp.int32)
    i = jnp.int32(0x5F3759DF) - lax.shift_right_logical(i, 1)
    y = lax.bitcast_convert_type(i, jnp.float32)
    for _ in range(3):
        y = y * (jnp.float32(1.5) - jnp.float32(0.5) * x * y * y)
    return y


def _body(ent, rel, iph_h, ipr_h, ipt_h, inh_h, inr_h, int_h, out,
          iph, ipr, ipt, inh, inr, itn, rpe, rne, rr, lv, sem_i,
          sem0, sem1, sem2, sem3):
    wid = lax.axis_index("s") * _NC + lax.axis_index("c")
    base = wid * _PW

    stage = [
        pltpu.async_copy(iph_h.at[pl.ds(base, _PW)], iph, sem_i),
        pltpu.async_copy(ipr_h.at[pl.ds(base, _PW)], ipr, sem_i),
        pltpu.async_copy(ipt_h.at[pl.ds(base, _PW)], ipt, sem_i),
        pltpu.async_copy(inh_h.at[pl.ds(base, _PW)], inh, sem_i),
        pltpu.async_copy(inr_h.at[pl.ds(base, _PW)], inr, sem_i),
        pltpu.async_copy(int_h.at[pl.ds(base, _PW)], itn, sem_i),
    ]
    for c in stage:
        c.wait()

    sems = (sem0, sem1, sem2, sem3)
    copies = []
    for ch in range(_CK):
        start, size = _CHUNKS[ch]
        isl = pl.ds(start, size)
        psl = pl.ds(start, size)
        nsl = pl.ds(_PW + start, size)
        copies.append((
            pltpu.async_copy(ent.at[iph.at[isl]], rpe.at[psl], sems[ch]),
            pltpu.async_copy(ent.at[ipt.at[isl]], rne.at[psl], sems[ch]),
            pltpu.async_copy(rel.at[ipr.at[isl]], rr.at[psl], sems[ch]),
            pltpu.async_copy(ent.at[inh.at[isl]], rpe.at[nsl], sems[ch]),
            pltpu.async_copy(ent.at[itn.at[isl]], rne.at[nsl], sems[ch]),
            pltpu.async_copy(rel.at[inr.at[isl]], rr.at[nsl], sems[ch]),
        ))

    lane = lax.iota(jnp.int32, 16)
    zero = jnp.zeros((16,), jnp.float32)
    two = jnp.float32(2.0)
    eps_n = jnp.float32(1e-24)
    eps_d = jnp.float32(1e-12)

    def dim_body(d, acc):
        (pbase, nbase,
         psh, psr, pst, pshr, psht, psrt,
         nsh, nsr, nst, nshr, nsht, nsrt) = acc
        dv = (jnp.full((16,), d, jnp.int32) + lane) & jnp.int32(_D - 1)
        hh = plsc.load_gather(rpe, [pbase, dv])
        rv = plsc.load_gather(rr, [pbase, dv])
        tt = plsc.load_gather(rne, [pbase, dv])
        psh = psh + hh * hh
        psr = psr + rv * rv
        pst = pst + tt * tt
        pshr = pshr + hh * rv
        psht = psht + hh * tt
        psrt = psrt + rv * tt
        hh = plsc.load_gather(rpe, [nbase, dv])
        rv = plsc.load_gather(rr, [nbase, dv])
        tt = plsc.load_gather(rne, [nbase, dv])
        nsh = nsh + hh * hh
        nsr = nsr + rv * rv
        nst = nst + tt * tt
        nshr = nshr + hh * rv
        nsht = nsht + hh * tt
        nsrt = nsrt + rv * tt
        return (pbase, nbase,
                psh, psr, pst, pshr, psht, psrt,
                nsh, nsr, nst, nshr, nsht, nsrt)

    def group_body(gg, loss):
        for ch in range(_CK):
            @pl.when(gg == jnp.int32(_WAIT_AT[ch]))
            def _wait():
                for cp in copies[ch]:
                    cp.wait()

        if True:
            pbase = lane + gg * 16
            nbase = pbase + jnp.int32(_PW)
            acc = lax.fori_loop(0, _D, dim_body,
                                (pbase, nbase) + (zero,) * 12, unroll=16)
            (psh, psr, pst, pshr, psht, psrt,
             nsh, nsr, nst, nshr, nsht, nsrt) = acc[2:]

            ih = _rsqrt(jnp.maximum(psh, eps_n))
            it = _rsqrt(jnp.maximum(pst, eps_n))
            sp = (psh * ih * ih + psr + pst * it * it
                  + two * (ih * pshr - ih * it * psht - it * psrt)) + eps_d
            ih = _rsqrt(jnp.maximum(nsh, eps_n))
            it = _rsqrt(jnp.maximum(nst, eps_n))
            sn = (nsh * ih * ih + nsr + nst * it * it
                  + two * (ih * nshr - ih * it * nsht - it * nsrt)) + eps_d
            dp = sp * _rsqrt(sp)
            dn = sn * _rsqrt(sn)
            return loss + jnp.maximum(dp - dn + jnp.float32(_MARGIN),
                                      jnp.float32(0.0))

    loss = lax.fori_loop(0, _PW // 16, group_body, zero)

    lv[...] = loss
    pltpu.sync_copy(lv, out.at[wid])


@jax.jit
def _transe_loss(entity_emb, relation_emb, iph, ipr, ipt, inh, inr, itn):
    mesh = plsc.VectorSubcoreMesh(core_axis_name="c", subcore_axis_name="s")
    f = pl.kernel(
        _body,
        out_type=jax.ShapeDtypeStruct((_NW, 16), jnp.float32),
        mesh=mesh,
        compiler_params=pltpu.CompilerParams(needs_layout_passes=False),
        scratch_types=[
            pltpu.VMEM((_PW,), jnp.int32),
            pltpu.VMEM((_PW,), jnp.int32),
            pltpu.VMEM((_PW,), jnp.int32),
            pltpu.VMEM((_PW,), jnp.int32),
            pltpu.VMEM((_PW,), jnp.int32),
            pltpu.VMEM((_PW,), jnp.int32),
            pltpu.VMEM((2 * _PW, _D), jnp.float32),
            pltpu.VMEM((2 * _PW, _D), jnp.float32),
            pltpu.VMEM((2 * _PW, _D), jnp.float32),
            pltpu.VMEM((16,), jnp.float32),
            pltpu.SemaphoreType.DMA,
            pltpu.SemaphoreType.DMA,
            pltpu.SemaphoreType.DMA,
            pltpu.SemaphoreType.DMA,
            pltpu.SemaphoreType.DMA,
        ],
    )
    partials = f(entity_emb, relation_emb, iph, ipr, ipt, inh, inr, itn)
    return jnp.sum(partials)


def kernel(entity_emb, relation_emb, pos_heads, pos_rels, pos_tails,
           neg_heads, neg_rels, neg_tails):
    idx = [x.astype(jnp.int32) for x in (pos_heads, pos_rels, pos_tails,
                                         neg_heads, neg_rels, neg_tails)]
    return _transe_loss(entity_emb, relation_emb, *idx)
